# Initial kernel scaffold; baseline (speedup 1.0000x reference)
import jax
import jax.numpy as jnp
from jax.experimental import pallas as pl


def kernel(depth_indices, embed_table, freq_bands, W, b, gamma, beta):
    raise NotImplementedError("write your pallas kernel here")


# speedup vs baseline: 7.5941x; 7.5941x over previous
import functools
import math

import jax
import jax.numpy as jnp
from jax import lax
from jax.experimental import pallas as pl
from jax.experimental.pallas import tpu as pltpu
from jax.experimental.pallas import tpu_sc as plsc

_EMBED = 64
_DEPTHS = 16
_NTOK = 819200


def _table_body(emb_ref, fb_ref, w1_ref, w2_ref, w3_ref, b_ref, g_ref,
                beta_ref, out_ref):
    depth = lax.broadcasted_iota(jnp.int32, (_DEPTHS, 1), 0).astype(
        jnp.float32) * (1.0 / (_DEPTHS - 1))
    ang = depth * fb_ref[...] * math.pi
    sin_f = jnp.sin(ang)
    cos_f = jnp.cos(ang)
    dn = (((1,), (1,)), ((), ()))
    y = lax.dot_general(emb_ref[...], w1_ref[...], dn,
                        preferred_element_type=jnp.float32)
    y = y + lax.dot_general(sin_f, w2_ref[...], dn,
                            preferred_element_type=jnp.float32)
    y = y + lax.dot_general(cos_f, w3_ref[...], dn,
                            preferred_element_type=jnp.float32)
    y = y + b_ref[...]
    mean = jnp.mean(y, axis=1, keepdims=True)
    var = jnp.mean((y - mean) ** 2, axis=1, keepdims=True)
    yn = (y - mean) * lax.rsqrt(var + 1e-5)
    yn = yn * g_ref[...] + beta_ref[...]
    enc = 0.5 * yn * (1.0 + lax.erf(yn * (1.0 / math.sqrt(2.0))))
    out_ref[...] = enc


def _build_table(embed_table, freq_bands, W, b, gamma, beta):
    w1 = W[:, : _EMBED // 2]
    w2 = W[:, _EMBED // 2: _EMBED // 2 + _EMBED // 4]
    w3 = W[:, _EMBED // 2 + _EMBED // 4:]
    return pl.pallas_call(
        _table_body,
        out_shape=jax.ShapeDtypeStruct((_DEPTHS, _EMBED), jnp.float32),
    )(embed_table, freq_bands.reshape(1, -1), w1, w2, w3,
      b.reshape(1, -1), gamma.reshape(1, -1), beta.reshape(1, -1))


_NC = 2
_NS = 16
_NW = _NC * _NS
_BLK = 640
_BLOCKS = _NTOK // _BLK
_BPW = _BLOCKS // _NW

_sc_mesh = plsc.VectorSubcoreMesh(core_axis_name="c", subcore_axis_name="s")


_LANES = 16
_CHUNKS = _EMBED // _LANES
_BW = _BLK * _EMBED


@functools.partial(
    pl.kernel,
    mesh=_sc_mesh,
    out_type=jax.ShapeDtypeStruct((_NTOK * _EMBED,), jnp.float32),
    scratch_types=[
        pltpu.VMEM((_DEPTHS * _EMBED,), jnp.float32),
        pltpu.VMEM((_BLK,), jnp.int32),
        pltpu.VMEM((_BLK,), jnp.int32),
        pltpu.VMEM((_BW,), jnp.float32),
        pltpu.VMEM((_BW,), jnp.float32),
        pltpu.SemaphoreType.DMA,
        pltpu.SemaphoreType.DMA,
        pltpu.SemaphoreType.DMA,
        pltpu.SemaphoreType.DMA,
    ],
    compiler_params=pltpu.CompilerParams(
        use_tc_tiling_on_sc=False, needs_layout_passes=False),
)
def _sc_gather(table_hbm, idx_hbm, out_hbm, tab_v, idx0, idx1, rows0, rows1,
               sw0, sw1, si0, si1):
    wid = lax.axis_index("s") * _NC + lax.axis_index("c")
    idx_v = [idx0, idx1]
    rows_v = [rows0, rows1]
    sem_w = [sw0, sw1]
    sem_i = [si0, si1]

    pltpu.sync_copy(table_hbm, tab_v)

    def expand_block(idx_ref, rows_ref):
        @plsc.parallel_loop(0, _BLK, step=_LANES)
        def _body(t0):
            srcs = idx_ref[pl.ds(t0, _LANES)] * _EMBED
            for k in range(_LANES):
                src = srcs[k]
                dst = (t0 + k) * _EMBED
                for c in range(_CHUNKS):
                    rows_ref[pl.ds(dst + c * _LANES, _LANES)] = (
                        tab_v[pl.ds(src + c * _LANES, _LANES)])

    def wait_write(b):
        pltpu.make_async_copy(
            rows_v[b], out_hbm.at[pl.ds(0, _BW)], sem_w[b]).wait()

    def wait_idx(b):
        pltpu.make_async_copy(
            idx_hbm.at[pl.ds(0, _BLK)], idx_v[b], sem_i[b]).wait()

    pltpu.async_copy(
        idx_hbm.at[pl.ds(wid * _BPW * _BLK, _BLK)], idx_v[0], sem_i[0])

    def superstep(s, carry):
        for b in range(2):
            i = s * 2 + b
            blk = wid * _BPW + i
            wait_idx(b)

            @pl.when(i + 1 < _BPW)
            def _():
                pltpu.async_copy(
                    idx_hbm.at[pl.ds((blk + 1) * _BLK, _BLK)],
                    idx_v[1 - b], sem_i[1 - b])

            @pl.when(s > 0)
            def _():
                wait_write(b)

            expand_block(idx_v[b], rows_v[b])
            pltpu.async_copy(
                rows_v[b], out_hbm.at[pl.ds(blk * _BW, _BW)], sem_w[b])
        return carry

    lax.fori_loop(0, _BPW // 2, superstep, 0)
    wait_write(0)
    wait_write(1)


def kernel(depth_indices, embed_table, freq_bands, W, b, gamma, beta):
    table = _build_table(embed_table, freq_bands, W, b, gamma, beta)
    flat = _sc_gather(table.reshape(-1), depth_indices.astype(jnp.int32))
    return flat.reshape(_NTOK, _EMBED)
